# baseline (speedup 1.0000x reference)
import jax
import jax.numpy as jnp
from jax import lax
from jax.experimental import pallas as pl
from jax.experimental.pallas import tpu as pltpu
from jax.experimental.pallas import tpu_sc as plsc

L = 64
MAXNODE = 512
MAXFANOUT = 8
N0 = 2 * L * MAXNODE * MAXFANOUT
NW0 = N0 // MAXFANOUT
NNZ = 2097152

NC = 2
NS = 16
NW = NC * NS
EPT = NNZ // NW
CH = 4096
NCH = EPT // CH
ACC_T = N0 // NS
DCH = 2048

_params = pltpu.CompilerParams(needs_layout_passes=False)
_mesh = plsc.VectorSubcoreMesh(core_axis_name="c", subcore_axis_name="s")


def _zero_acc(pq, acc_s, s):
    def zstep(i, carry):
        pq[pl.ds(i * 16, 16)] = jnp.zeros((16,), jnp.float32)
        return carry
    lax.fori_loop(0, CH // 16, zstep, 0)
    for t in range(ACC_T // CH):
        pltpu.sync_copy(pq, acc_s.at[pl.ds(s * ACC_T + t * CH, CH)])


def _flatten_gidx(dims, xs, out, n):
    def fstep(i, carry):
        sl = pl.ds(i * 16, 16)
        g = xs[0][sl]
        for d, x in zip(dims[1:], xs[1:]):
            g = g * d + x[sl]
        out[sl] = g
        return carry
    lax.fori_loop(0, n // 16, fstep, 0)


def _k1_body(ai0_h, ai1_h, ai2_h, ai3_h, ai4_h, aval_h, w_h, out_h,
             tbl_s, acc_s, x1, x2, x3, x4, sq, vq, wq, pq, sem):
    c = lax.axis_index("c")
    s = lax.axis_index("s")
    wid = c * NS + s

    pltpu.sync_copy(w_h.at[pl.ds(s * (N0 // NS), N0 // NS)],
                    tbl_s.at[pl.ds(s * (N0 // NS), N0 // NS)])
    _zero_acc(pq, acc_s, s)
    plsc.subcore_barrier()

    def chunk(j, carry):
        base = wid * EPT + j * CH
        pltpu.sync_copy(ai0_h.at[pl.ds(base, CH)], sq)
        pltpu.sync_copy(ai1_h.at[pl.ds(base, CH)], x1)
        pltpu.sync_copy(ai2_h.at[pl.ds(base, CH)], x2)
        pltpu.sync_copy(ai3_h.at[pl.ds(base, CH)], x3)
        pltpu.sync_copy(ai4_h.at[pl.ds(base, CH)], x4)
        pltpu.sync_copy(aval_h.at[pl.ds(base, CH)], vq)
        _flatten_gidx((2, L, MAXNODE, MAXFANOUT), (x1, x2, x3, x4), x1, CH)
        pltpu.async_copy(tbl_s.at[x1], wq, sem).wait()

        def mstep(i, c2):
            sl = pl.ds(i * 16, 16)
            pq[sl] = wq[sl] * vq[sl]
            return c2
        lax.fori_loop(0, CH // 16, mstep, 0)
        pltpu.sync_copy(pq, acc_s.at[sq], add=True)
        return carry
    lax.fori_loop(0, NCH, chunk, 0)
    plsc.subcore_barrier()

    pltpu.sync_copy(acc_s.at[pl.ds(s * ACC_T, ACC_T)],
                    out_h.at[pl.ds(c * N0 + s * ACC_T, ACC_T)])


_k1 = pl.kernel(
    _k1_body,
    out_type=jax.ShapeDtypeStruct((NC * N0,), jnp.float32),
    mesh=_mesh,
    compiler_params=_params,
    scratch_types=[
        pltpu.VMEM_SHARED((N0,), jnp.float32),
        pltpu.VMEM_SHARED((N0,), jnp.float32),
        pltpu.VMEM((CH,), jnp.int32),
        pltpu.VMEM((CH,), jnp.int32),
        pltpu.VMEM((CH,), jnp.int32),
        pltpu.VMEM((CH,), jnp.int32),
        pltpu.VMEM((CH,), jnp.int32),
        pltpu.VMEM((CH,), jnp.float32),
        pltpu.VMEM((CH,), jnp.float32),
        pltpu.VMEM((CH,), jnp.float32),
        pltpu.SemaphoreType.DMA,
    ],
)


def _k2_body(wi0_h, wi1_h, wi2_h, wi3_h, wval_h, p_h, ld_h, q_h, o_h, wl_h,
             acc_s, wl_s, tbl_r, x1, x2, x3, sq, vq, pq, oq, sem):
    c = lax.axis_index("c")
    s = lax.axis_index("s")
    wid = c * NS + s
    iota = lax.iota(jnp.int32, 16)

    _zero_acc(pq, acc_s, s)

    def dsub(t, carry):
        off = s * ACC_T + t * DCH
        off8 = s * (ACC_T // MAXFANOUT) + t * (DCH // MAXFANOUT)
        pltpu.sync_copy(p_h.at[pl.ds(off, DCH)], vq.at[pl.ds(0, DCH)])
        pltpu.sync_copy(p_h.at[pl.ds(N0 + off, DCH)], pq.at[pl.ds(0, DCH)])

        def astep(i, c2):
            sl = pl.ds(i * 16, 16)
            oq[sl] = vq[sl] + pq[sl]
            return c2
        lax.fori_loop(0, DCH // 16, astep, 0)

        @pl.when(c == 0)
        def _():
            pltpu.sync_copy(oq, o_h.at[pl.ds(off, DCH)])
        pltpu.sync_copy(ld_h.at[pl.ds(off, DCH)], vq.at[pl.ds(0, DCH)])

        def mstep(i, c2):
            sl = pl.ds(i * 16, 16)
            pq[sl] = vq[sl] * oq[sl]
            return c2
        lax.fori_loop(0, DCH // 16, mstep, 0)

        def gstep(i, c2):
            w = jnp.zeros((16,), jnp.float32)
            for t2 in range(MAXFANOUT):
                w = w + plsc.load_gather(
                    pq, [i * 128 + iota * MAXFANOUT + t2])
            oq[pl.ds(i * 16, 16)] = w
            return c2
        lax.fori_loop(0, DCH // 128, gstep, 0)
        pltpu.sync_copy(oq.at[pl.ds(0, DCH // MAXFANOUT)],
                        wl_s.at[pl.ds(off8, DCH // MAXFANOUT)])
        return carry
    lax.fori_loop(0, ACC_T // DCH, dsub, 0)
    plsc.subcore_barrier()

    pltpu.sync_copy(wl_s, tbl_r)

    @pl.when(c == 0)
    def _():
        pltpu.sync_copy(wl_s.at[pl.ds(s * (NW0 // NS), NW0 // NS)],
                        wl_h.at[pl.ds(s * (NW0 // NS), NW0 // NS)])

    def chunk(j, carry):
        base = wid * EPT + j * CH
        pltpu.sync_copy(wi0_h.at[pl.ds(base, CH)], sq)
        pltpu.sync_copy(wi1_h.at[pl.ds(base, CH)], x1)
        pltpu.sync_copy(wi2_h.at[pl.ds(base, CH)], x2)
        pltpu.sync_copy(wi3_h.at[pl.ds(base, CH)], x3)
        pltpu.sync_copy(wval_h.at[pl.ds(base, CH)], vq)
        _flatten_gidx((2, L, MAXNODE), (x1, x2, x3), x1, CH)

        def mstep(i, c2):
            sl = pl.ds(i * 16, 16)
            w = plsc.load_gather(tbl_r, [x1[sl]])
            pq[sl] = w * vq[sl]
            return c2
        lax.fori_loop(0, CH // 16, mstep, 0)
        pltpu.sync_copy(pq, acc_s.at[sq], add=True)
        return carry
    lax.fori_loop(0, NCH, chunk, 0)
    plsc.subcore_barrier()

    pltpu.sync_copy(acc_s.at[pl.ds(s * ACC_T, ACC_T)],
                    q_h.at[pl.ds(c * N0 + s * ACC_T, ACC_T)])


_k2 = pl.kernel(
    _k2_body,
    out_type=[jax.ShapeDtypeStruct((NC * N0,), jnp.float32),
              jax.ShapeDtypeStruct((N0,), jnp.float32),
              jax.ShapeDtypeStruct((NW0,), jnp.float32)],
    mesh=_mesh,
    compiler_params=_params,
    scratch_types=[
        pltpu.VMEM_SHARED((N0,), jnp.float32),
        pltpu.VMEM_SHARED((NW0,), jnp.float32),
        pltpu.VMEM((NW0,), jnp.float32),
        pltpu.VMEM((CH,), jnp.int32),
        pltpu.VMEM((CH,), jnp.int32),
        pltpu.VMEM((CH,), jnp.int32),
        pltpu.VMEM((CH,), jnp.int32),
        pltpu.VMEM((CH,), jnp.float32),
        pltpu.VMEM((CH,), jnp.float32),
        pltpu.VMEM((DCH,), jnp.float32),
        pltpu.SemaphoreType.DMA,
    ],
)


def _group_sum_mat():
    l = lax.broadcasted_iota(jnp.int32, (128, 16), 0)
    g = lax.broadcasted_iota(jnp.int32, (128, 16), 1)
    return (l // 8 == g).astype(jnp.float32)


def _dense2_body(q0, q1, o, wl, out_ref):
    lw = (q0[...] + q1[...]) * o[...]
    out_ref[...] = wl[...] + jnp.dot(lw, _group_sum_mat(),
                                     preferred_element_type=jnp.float32)


_R = N0 // 128
_BR = 512
_G = 8


def _dense2(q0, q1, o, wl):
    big = pl.BlockSpec((_BR, 128), lambda i: (i, 0))
    small = pl.BlockSpec((_BR, 16), lambda i: (i, 0))
    return pl.pallas_call(
        _dense2_body,
        grid=(_G,),
        in_specs=[big, big, big, small],
        out_specs=small,
        out_shape=jax.ShapeDtypeStruct((_R, 16), jnp.float32),
    )(q0, q1, o, wl)


@jax.jit
def kernel(weight, load, adj_indices, adj_values, wire_indices, wire_values):
    p = _k1(adj_indices[0], adj_indices[1], adj_indices[2], adj_indices[3],
            adj_indices[4], adj_values, weight.reshape(-1))
    q, o, wl = _k2(wire_indices[0], wire_indices[1], wire_indices[2],
                   wire_indices[3], wire_values, p, load.reshape(-1))
    q = q.reshape(NC, _R, 128)
    out = _dense2(q[0], q[1], o.reshape(_R, 128), wl.reshape(_R, 16))
    return out.reshape(2, L, MAXNODE)


# speedup vs baseline: 1.0078x; 1.0078x over previous
import jax
import jax.numpy as jnp
from jax import lax
from jax.experimental import pallas as pl
from jax.experimental.pallas import tpu as pltpu
from jax.experimental.pallas import tpu_sc as plsc

L = 64
MAXNODE = 512
MAXFANOUT = 8
N0 = 2 * L * MAXNODE * MAXFANOUT
NW0 = N0 // MAXFANOUT
NNZ = 2097152

NC = 2
NS = 16
CH = 4096
EPT1 = NNZ // NS
NCH1 = EPT1 // CH
EPT2 = NNZ // (NC * NS)
NCH2 = EPT2 // CH
ACC_T = N0 // NS
DCH = 2048

_params = pltpu.CompilerParams(needs_layout_passes=False)
_mesh = plsc.VectorSubcoreMesh(core_axis_name="c", subcore_axis_name="s")


def _sc_body(ai0, ai1, ai2, ai3, ai4, aval, w_h, wi0, wi1, wi2, wi3, wval,
             ld_h, q_h, o_h, wl_h,
             tbl_s, acc_s, wl_s,
             x1a, x2a, x3a, x4a, sqa, vqa,
             x1b, x2b, x3b, x4b, sqb, vqb,
             wq, pq, oq, sema, semb, gsem):
    c = lax.axis_index("c")
    s = lax.axis_index("s")
    iota = lax.iota(jnp.int32, 16)

    def vloop(n, f):
        def step(i, carry):
            f(pl.ds(i * 16, 16))
            return carry
        lax.fori_loop(0, n // 16, step, 0)

    def zero_acc():
        def zf(sl):
            pq[sl] = jnp.zeros((16,), jnp.float32)
        vloop(CH, zf)
        for t in range(ACC_T // CH):
            pltpu.sync_copy(pq, acc_s.at[pl.ds(s * ACC_T + t * CH, CH)])

    bufs = ((x1a, x2a, x3a, x4a, sqa, vqa, sema),
            (x1b, x2b, x3b, x4b, sqb, vqb, semb))

    def start_in(rows, vals, base, nrows, b):
        x1, x2, x3, x4, sq, vq, sem = bufs[b]
        dsts = (x1, x2, x3, x4)[:nrows]
        pltpu.async_copy(rows[0].at[pl.ds(base, CH)], sq, sem)
        for r, d in zip(rows[1:], dsts):
            pltpu.async_copy(r.at[pl.ds(base, CH)], d, sem)
        pltpu.async_copy(vals.at[pl.ds(base, CH)], vq, sem)

    def wait_in(rows, vals, nrows, b):
        x1, x2, x3, x4, sq, vq, sem = bufs[b]
        dsts = (x1, x2, x3, x4)[:nrows]
        pltpu.make_async_copy(rows[0].at[pl.ds(0, CH)], sq, sem).wait()
        for r, d in zip(rows[1:], dsts):
            pltpu.make_async_copy(r.at[pl.ds(0, CH)], d, sem).wait()
        pltpu.make_async_copy(vals.at[pl.ds(0, CH)], vq, sem).wait()

    def work(dims, nrows, b):
        x1, x2, x3, x4, sq, vq, _ = bufs[b]
        xs = (x1, x2, x3, x4)[:nrows]

        def ff(sl):
            g = xs[0][sl]
            for d, x in zip(dims[1:], xs[1:]):
                g = g * d + x[sl]
            x1[sl] = g
        vloop(CH, ff)
        src = tbl_s if nrows == 4 else wl_s
        pltpu.async_copy(src.at[x1], wq, gsem).wait()

        def mf(sl):
            pq[sl] = wq[sl] * vq[sl]
        vloop(CH, mf)
        pltpu.sync_copy(pq, acc_s.at[sq], add=True)

    def sparse_phase_sync(rows, vals, dims, tile_base, nch):
        nrows = len(dims)
        x1, x2, x3, x4, sq, vq, _ = bufs[0]
        dsts = (x1, x2, x3, x4)[:nrows]

        def one(j, carry):
            base = tile_base + j * CH
            pltpu.sync_copy(rows[0].at[pl.ds(base, CH)], sq)
            for r, d in zip(rows[1:], dsts):
                pltpu.sync_copy(r.at[pl.ds(base, CH)], d)
            pltpu.sync_copy(vals.at[pl.ds(base, CH)], vq)
            work(dims, nrows, 0)
            return carry
        lax.fori_loop(0, nch, one, 0)

    def sparse_phase(rows, vals, dims, tile_base, nch):
        nrows = len(dims)
        start_in(rows, vals, tile_base, nrows, 0)

        def pair(t, carry):
            j = 2 * t
            wait_in(rows, vals, nrows, 0)
            start_in(rows, vals, tile_base + (j + 1) * CH, nrows, 1)
            work(dims, nrows, 0)
            wait_in(rows, vals, nrows, 1)

            @pl.when(j + 2 < nch)
            def _():
                start_in(rows, vals, tile_base + (j + 2) * CH, nrows, 0)
            work(dims, nrows, 1)
            return carry
        lax.fori_loop(0, nch // 2, pair, 0)

    pltpu.sync_copy(w_h.at[pl.ds(s * (N0 // NS), N0 // NS)],
                    tbl_s.at[pl.ds(s * (N0 // NS), N0 // NS)])
    zero_acc()
    plsc.subcore_barrier()

    sparse_phase((ai0, ai1, ai2, ai3, ai4), aval,
                 (2, L, MAXNODE, MAXFANOUT), s * EPT1, NCH1)
    plsc.subcore_barrier()

    def dsub(t, carry):
        off = s * ACC_T + t * DCH
        off8 = s * (ACC_T // MAXFANOUT) + t * (DCH // MAXFANOUT)
        pltpu.sync_copy(acc_s.at[pl.ds(off, DCH)], vqa.at[pl.ds(0, DCH)])

        @pl.when(c == 0)
        def _():
            pltpu.sync_copy(vqa.at[pl.ds(0, DCH)], o_h.at[pl.ds(off, DCH)])
        pltpu.sync_copy(ld_h.at[pl.ds(off, DCH)], wq.at[pl.ds(0, DCH)])

        def mf(sl):
            pq[sl] = vqa[sl] * wq[sl]
        vloop(DCH, mf)

        def gstep(i, c2):
            w = jnp.zeros((16,), jnp.float32)
            for t2 in range(MAXFANOUT):
                w = w + plsc.load_gather(
                    pq, [i * 128 + iota * MAXFANOUT + t2])
            oq[pl.ds(i * 16, 16)] = w
            return c2
        lax.fori_loop(0, DCH // 128, gstep, 0)
        pltpu.sync_copy(oq.at[pl.ds(0, DCH // MAXFANOUT)],
                        wl_s.at[pl.ds(off8, DCH // MAXFANOUT)])
        return carry
    lax.fori_loop(0, ACC_T // DCH, dsub, 0)
    zero_acc()
    plsc.subcore_barrier()

    @pl.when(c == 0)
    def _():
        pltpu.sync_copy(wl_s.at[pl.ds(s * (NW0 // NS), NW0 // NS)],
                        wl_h.at[pl.ds(s * (NW0 // NS), NW0 // NS)])

    sparse_phase((wi0, wi1, wi2, wi3), wval,
                 (2, L, MAXNODE), (c * NS + s) * EPT2, NCH2)
    plsc.subcore_barrier()

    pltpu.sync_copy(acc_s.at[pl.ds(s * ACC_T, ACC_T)],
                    q_h.at[pl.ds(c * N0 + s * ACC_T, ACC_T)])


_sc_scratch = (
    [pltpu.VMEM_SHARED((N0,), jnp.float32),
     pltpu.VMEM_SHARED((N0,), jnp.float32),
     pltpu.VMEM_SHARED((NW0,), jnp.float32)]
    + [pltpu.VMEM((CH,), jnp.int32)] * 5
    + [pltpu.VMEM((CH,), jnp.float32)]
    + [pltpu.VMEM((CH,), jnp.int32)] * 5
    + [pltpu.VMEM((CH,), jnp.float32)]
    + [pltpu.VMEM((CH,), jnp.float32),
       pltpu.VMEM((CH,), jnp.float32),
       pltpu.VMEM((DCH,), jnp.float32),
       pltpu.SemaphoreType.DMA,
       pltpu.SemaphoreType.DMA,
       pltpu.SemaphoreType.DMA]
)

_sc = pl.kernel(
    _sc_body,
    out_type=[jax.ShapeDtypeStruct((NC * N0,), jnp.float32),
              jax.ShapeDtypeStruct((N0,), jnp.float32),
              jax.ShapeDtypeStruct((NW0,), jnp.float32)],
    mesh=_mesh,
    compiler_params=_params,
    scratch_types=_sc_scratch,
)


def _group_sum_mat():
    l = lax.broadcasted_iota(jnp.int32, (128, 16), 0)
    g = lax.broadcasted_iota(jnp.int32, (128, 16), 1)
    return (l // 8 == g).astype(jnp.float32)


def _dense2_body(q0, q1, o, wl, out_ref):
    lw = (q0[...] + q1[...]) * o[...]
    out_ref[...] = wl[...] + jnp.dot(lw, _group_sum_mat(),
                                     preferred_element_type=jnp.float32)


_R = N0 // 128
_BR = 512
_G = 8


def _dense2(q0, q1, o, wl):
    big = pl.BlockSpec((_BR, 128), lambda i: (i, 0))
    small = pl.BlockSpec((_BR, 16), lambda i: (i, 0))
    return pl.pallas_call(
        _dense2_body,
        grid=(_G,),
        in_specs=[big, big, big, small],
        out_specs=small,
        out_shape=jax.ShapeDtypeStruct((_R, 16), jnp.float32),
    )(q0, q1, o, wl)


@jax.jit
def kernel(weight, load, adj_indices, adj_values, wire_indices, wire_values):
    q, o, wl = _sc(adj_indices[0], adj_indices[1], adj_indices[2],
                   adj_indices[3], adj_indices[4], adj_values,
                   weight.reshape(-1),
                   wire_indices[0], wire_indices[1], wire_indices[2],
                   wire_indices[3], wire_values, load.reshape(-1))
    q = q.reshape(NC, _R, 128)
    out = _dense2(q[0], q[1], o.reshape(_R, 128), wl.reshape(_R, 16))
    return out.reshape(2, L, MAXNODE)


# speedup vs baseline: 1.6188x; 1.6063x over previous
import jax
import jax.numpy as jnp
from jax import lax
from jax.experimental import pallas as pl
from jax.experimental.pallas import tpu as pltpu
from jax.experimental.pallas import tpu_sc as plsc

L = 64
MAXNODE = 512
MAXFANOUT = 8
N0 = 2 * L * MAXNODE * MAXFANOUT
NNZ = 2097152

NC = 2
NS = 16
NW = NC * NS
EPT = NNZ // NW
CH = 4096
NCH = EPT // CH
ACC_T = N0 // NS

_params = pltpu.CompilerParams(needs_layout_passes=False)
_mesh = plsc.VectorSubcoreMesh(core_axis_name="c", subcore_axis_name="s")


def _make_sc_scatter(table_size: int, table_in_vmem: bool):
    tslice = table_size // NS

    def body(gidx_h, sidx_h, val_h, tbl_h, out_h,
             acc_s, tbl_r, gqa, sqa, vqa, gqb, sqb, vqb, pq,
             sema, semb, gsem, *maybe_wq):
        c = lax.axis_index("c")
        s = lax.axis_index("s")
        wid = c * NS + s
        bufs = ((gqa, sqa, vqa, sema), (gqb, sqb, vqb, semb))

        def vloop(n, f):
            def step(i, carry):
                f(pl.ds(i * 16, 16))
                return carry
            lax.fori_loop(0, n // 16, step, 0)

        def zf(sl):
            pq[sl] = jnp.zeros((16,), jnp.float32)
        vloop(CH, zf)
        for t in range(ACC_T // CH):
            pltpu.sync_copy(pq, acc_s.at[pl.ds(s * ACC_T + t * CH, CH)])
        if table_in_vmem:
            pltpu.sync_copy(tbl_h, tbl_r)
        else:
            pltpu.sync_copy(tbl_h.at[pl.ds(s * tslice, tslice)],
                            tbl_r.at[pl.ds(s * tslice, tslice)])
        plsc.subcore_barrier()

        def start_in(j, b):
            gq, sq, vq, sem = bufs[b]
            base = wid * EPT + j * CH
            pltpu.async_copy(gidx_h.at[pl.ds(base, CH)], gq, sem)
            pltpu.async_copy(sidx_h.at[pl.ds(base, CH)], sq, sem)
            pltpu.async_copy(val_h.at[pl.ds(base, CH)], vq, sem)

        def wait_in(b):
            gq, sq, vq, sem = bufs[b]
            pltpu.make_async_copy(gidx_h.at[pl.ds(0, CH)], gq, sem).wait()
            pltpu.make_async_copy(sidx_h.at[pl.ds(0, CH)], sq, sem).wait()
            pltpu.make_async_copy(val_h.at[pl.ds(0, CH)], vq, sem).wait()

        def work(b):
            gq, sq, vq, _ = bufs[b]
            if table_in_vmem:
                def mf(sl):
                    w = plsc.load_gather(tbl_r, [gq[sl]])
                    pq[sl] = w * vq[sl]
            else:
                wq = maybe_wq[0]
                pltpu.async_copy(tbl_r.at[gq], wq, gsem).wait()

                def mf(sl):
                    pq[sl] = wq[sl] * vq[sl]
            vloop(CH, mf)
            pltpu.sync_copy(pq, acc_s.at[sq], add=True)

        start_in(0, 0)

        def pair(t, carry):
            j = 2 * t
            wait_in(0)
            start_in(j + 1, 1)
            work(0)
            wait_in(1)

            @pl.when(j + 2 < NCH)
            def _():
                start_in(j + 2, 0)
            work(1)
            return carry
        lax.fori_loop(0, NCH // 2, pair, 0)
        plsc.subcore_barrier()

        pltpu.sync_copy(acc_s.at[pl.ds(s * ACC_T, ACC_T)],
                        out_h.at[pl.ds(c * N0 + s * ACC_T, ACC_T)])

    tbl_scratch = (pltpu.VMEM((table_size,), jnp.float32) if table_in_vmem
                   else pltpu.VMEM_SHARED((table_size,), jnp.float32))
    scratch = [
        pltpu.VMEM_SHARED((N0,), jnp.float32),
        tbl_scratch,
        pltpu.VMEM((CH,), jnp.int32),
        pltpu.VMEM((CH,), jnp.int32),
        pltpu.VMEM((CH,), jnp.float32),
        pltpu.VMEM((CH,), jnp.int32),
        pltpu.VMEM((CH,), jnp.int32),
        pltpu.VMEM((CH,), jnp.float32),
        pltpu.VMEM((CH,), jnp.float32),
        pltpu.SemaphoreType.DMA,
        pltpu.SemaphoreType.DMA,
        pltpu.SemaphoreType.DMA,
    ]
    if not table_in_vmem:
        scratch.append(pltpu.VMEM((CH,), jnp.float32))
    return pl.kernel(
        body,
        out_type=jax.ShapeDtypeStruct((NC * N0,), jnp.float32),
        mesh=_mesh,
        compiler_params=_params,
        scratch_types=scratch,
    )


def _group_sum_mat():
    l = lax.broadcasted_iota(jnp.int32, (128, 16), 0)
    g = lax.broadcasted_iota(jnp.int32, (128, 16), 1)
    return (l // 8 == g).astype(jnp.float32)


def _dense1_body(p0, p1, ld, o_ref, wl_ref):
    o = p0[...] + p1[...]
    o_ref[...] = o
    wl_ref[...] = jnp.dot(ld[...] * o, _group_sum_mat(),
                          preferred_element_type=jnp.float32)


def _dense2_body(q0, q1, o, wl, out_ref):
    lw = (q0[...] + q1[...]) * o[...]
    out_ref[...] = wl[...] + jnp.dot(lw, _group_sum_mat(),
                                     preferred_element_type=jnp.float32)


_R = N0 // 128
_BR = 512
_G = 8


def _dense1(p0, p1, ld):
    big = pl.BlockSpec((_BR, 128), lambda i: (i, 0))
    small = pl.BlockSpec((_BR, 16), lambda i: (i, 0))
    return pl.pallas_call(
        _dense1_body,
        grid=(_G,),
        in_specs=[big, big, big],
        out_specs=[big, small],
        out_shape=[jax.ShapeDtypeStruct((_R, 128), jnp.float32),
                   jax.ShapeDtypeStruct((_R, 16), jnp.float32)],
    )(p0, p1, ld)


def _dense2(q0, q1, o, wl):
    big = pl.BlockSpec((_BR, 128), lambda i: (i, 0))
    small = pl.BlockSpec((_BR, 16), lambda i: (i, 0))
    return pl.pallas_call(
        _dense2_body,
        grid=(_G,),
        in_specs=[big, big, big, small],
        out_specs=small,
        out_shape=jax.ShapeDtypeStruct((_R, 16), jnp.float32),
    )(q0, q1, o, wl)


@jax.jit
def kernel(weight, load, adj_indices, adj_values, wire_indices, wire_values):
    gidx = ((adj_indices[1] * L + adj_indices[2]) * MAXNODE
            + adj_indices[3]) * MAXFANOUT + adj_indices[4]
    wgidx = (wire_indices[1] * L + wire_indices[2]) * MAXNODE + wire_indices[3]

    sc1 = _make_sc_scatter(N0, table_in_vmem=False)
    p = sc1(gidx, adj_indices[0], adj_values, weight.reshape(-1))
    p = p.reshape(NC, _R, 128)
    o, wl = _dense1(p[0], p[1], load.reshape(_R, 128))

    sc2 = _make_sc_scatter(N0 // MAXFANOUT, table_in_vmem=True)
    q = sc2(wgidx, wire_indices[0], wire_values, wl.reshape(-1))
    q = q.reshape(NC, _R, 128)
    out = _dense2(q[0], q[1], o, wl)
    return out.reshape(2, L, MAXNODE)


# speedup vs baseline: 1.6393x; 1.0127x over previous
import jax
import jax.numpy as jnp
from jax import lax
from jax.experimental import pallas as pl
from jax.experimental.pallas import tpu as pltpu
from jax.experimental.pallas import tpu_sc as plsc

L = 64
MAXNODE = 512
MAXFANOUT = 8
N0 = 2 * L * MAXNODE * MAXFANOUT
NNZ = 2097152

NC = 2
NS = 16
NW = NC * NS
EPT = NNZ // NW
CH = 4096
NCH = EPT // CH
ACC_T = N0 // NS

_params = pltpu.CompilerParams(needs_layout_passes=False)
_mesh = plsc.VectorSubcoreMesh(core_axis_name="c", subcore_axis_name="s")


def _make_sc_scatter(table_size: int, table_in_vmem: bool):
    tslice = table_size // NS

    def body(gidx_h, sidx_h, val_h, tbl_h, out_h,
             acc_s, tbl_r, gqa, sqa, vqa, gqb, sqb, vqb, pq,
             sema, semb, gsem, *maybe_wq):
        c = lax.axis_index("c")
        s = lax.axis_index("s")
        wid = c * NS + s
        bufs = ((gqa, sqa, vqa, sema), (gqb, sqb, vqb, semb))

        def vloop(n, f):
            def step(i, carry):
                f(pl.ds(i * 16, 16))
                return carry
            lax.fori_loop(0, n // 16, step, 0)

        def zf(sl):
            pq[sl] = jnp.zeros((16,), jnp.float32)
        vloop(CH, zf)
        for t in range(ACC_T // CH):
            pltpu.sync_copy(pq, acc_s.at[pl.ds(s * ACC_T + t * CH, CH)])
        if table_in_vmem:
            pltpu.sync_copy(tbl_h, tbl_r)
        else:
            pltpu.sync_copy(tbl_h.at[pl.ds(s * tslice, tslice)],
                            tbl_r.at[pl.ds(s * tslice, tslice)])
        plsc.subcore_barrier()

        def start_in(j, b):
            gq, sq, vq, sem = bufs[b]
            base = wid * EPT + j * CH
            pltpu.async_copy(gidx_h.at[pl.ds(base, CH)], gq, sem)
            pltpu.async_copy(sidx_h.at[pl.ds(base, CH)], sq, sem)
            pltpu.async_copy(val_h.at[pl.ds(base, CH)], vq, sem)

        def wait_in(b):
            gq, sq, vq, sem = bufs[b]
            pltpu.make_async_copy(gidx_h.at[pl.ds(0, CH)], gq, sem).wait()
            pltpu.make_async_copy(sidx_h.at[pl.ds(0, CH)], sq, sem).wait()
            pltpu.make_async_copy(val_h.at[pl.ds(0, CH)], vq, sem).wait()

        start_in(0, 0)

        if table_in_vmem:
            def work(b):
                gq, sq, vq, _ = bufs[b]

                def mf(sl):
                    w = plsc.load_gather(tbl_r, [gq[sl]])
                    pq[sl] = w * vq[sl]
                vloop(CH, mf)
                pltpu.sync_copy(pq, acc_s.at[sq], add=True)

            def pair(t, carry):
                j = 2 * t
                wait_in(0)
                start_in(j + 1, 1)
                work(0)
                wait_in(1)

                @pl.when(j + 2 < NCH)
                def _():
                    start_in(j + 2, 0)
                work(1)
                return carry
            lax.fori_loop(0, NCH // 2, pair, 0)
        else:
            wq, pq2, ssa, ssb = maybe_wq
            pqs = (pq, pq2)
            ssems = (ssa, ssb)

            def gp(b):
                gq, sq, vq, _ = bufs[b]
                pltpu.async_copy(tbl_r.at[gq], wq, gsem).wait()
                dst = pqs[b]

                def mf(sl):
                    dst[sl] = wq[sl] * vq[sl]
                vloop(CH, mf)

            def sc_start(b):
                pltpu.async_copy(pqs[b], acc_s.at[bufs[b][1]], ssems[b],
                                 add=True)

            def sc_wait(b):
                pltpu.make_async_copy(pqs[b], acc_s.at[bufs[b][1]],
                                      ssems[b]).wait()

            def pair(t, carry):
                j = 2 * t
                wait_in(0)

                @pl.when(t > 0)
                def _():
                    sc_wait(1)
                start_in(j + 1, 1)
                gp(0)
                sc_start(0)
                wait_in(1)
                gp(1)
                sc_wait(0)

                @pl.when(j + 2 < NCH)
                def _():
                    start_in(j + 2, 0)
                sc_start(1)
                return carry
            lax.fori_loop(0, NCH // 2, pair, 0)
            sc_wait(1)
        plsc.subcore_barrier()

        pltpu.sync_copy(acc_s.at[pl.ds(s * ACC_T, ACC_T)],
                        out_h.at[pl.ds(c * N0 + s * ACC_T, ACC_T)])

    tbl_scratch = (pltpu.VMEM((table_size,), jnp.float32) if table_in_vmem
                   else pltpu.VMEM_SHARED((table_size,), jnp.float32))
    scratch = [
        pltpu.VMEM_SHARED((N0,), jnp.float32),
        tbl_scratch,
        pltpu.VMEM((CH,), jnp.int32),
        pltpu.VMEM((CH,), jnp.int32),
        pltpu.VMEM((CH,), jnp.float32),
        pltpu.VMEM((CH,), jnp.int32),
        pltpu.VMEM((CH,), jnp.int32),
        pltpu.VMEM((CH,), jnp.float32),
        pltpu.VMEM((CH,), jnp.float32),
        pltpu.SemaphoreType.DMA,
        pltpu.SemaphoreType.DMA,
        pltpu.SemaphoreType.DMA,
    ]
    if not table_in_vmem:
        scratch.extend([pltpu.VMEM((CH,), jnp.float32),
                        pltpu.VMEM((CH,), jnp.float32),
                        pltpu.SemaphoreType.DMA,
                        pltpu.SemaphoreType.DMA])
    return pl.kernel(
        body,
        out_type=jax.ShapeDtypeStruct((NC * N0,), jnp.float32),
        mesh=_mesh,
        compiler_params=_params,
        scratch_types=scratch,
    )


def _group_sum_mat():
    l = lax.broadcasted_iota(jnp.int32, (128, 16), 0)
    g = lax.broadcasted_iota(jnp.int32, (128, 16), 1)
    return (l // 8 == g).astype(jnp.float32)


def _dense1_body(p0, p1, ld, o_ref, wl_ref):
    o = p0[...] + p1[...]
    o_ref[...] = o
    wl_ref[...] = jnp.dot(ld[...] * o, _group_sum_mat(),
                          preferred_element_type=jnp.float32)


def _dense2_body(q0, q1, o, wl, out_ref):
    lw = (q0[...] + q1[...]) * o[...]
    out_ref[...] = wl[...] + jnp.dot(lw, _group_sum_mat(),
                                     preferred_element_type=jnp.float32)


_R = N0 // 128
_BR = 512
_G = 8


def _dense1(p0, p1, ld):
    big = pl.BlockSpec((_BR, 128), lambda i: (i, 0))
    small = pl.BlockSpec((_BR, 16), lambda i: (i, 0))
    return pl.pallas_call(
        _dense1_body,
        grid=(_G,),
        in_specs=[big, big, big],
        out_specs=[big, small],
        out_shape=[jax.ShapeDtypeStruct((_R, 128), jnp.float32),
                   jax.ShapeDtypeStruct((_R, 16), jnp.float32)],
    )(p0, p1, ld)


def _dense2(q0, q1, o, wl):
    big = pl.BlockSpec((_BR, 128), lambda i: (i, 0))
    small = pl.BlockSpec((_BR, 16), lambda i: (i, 0))
    return pl.pallas_call(
        _dense2_body,
        grid=(_G,),
        in_specs=[big, big, big, small],
        out_specs=small,
        out_shape=jax.ShapeDtypeStruct((_R, 16), jnp.float32),
    )(q0, q1, o, wl)


@jax.jit
def kernel(weight, load, adj_indices, adj_values, wire_indices, wire_values):
    gidx = ((adj_indices[1] * L + adj_indices[2]) * MAXNODE
            + adj_indices[3]) * MAXFANOUT + adj_indices[4]
    wgidx = (wire_indices[1] * L + wire_indices[2]) * MAXNODE + wire_indices[3]

    sc1 = _make_sc_scatter(N0, table_in_vmem=False)
    p = sc1(gidx, adj_indices[0], adj_values, weight.reshape(-1))
    p = p.reshape(NC, _R, 128)
    o, wl = _dense1(p[0], p[1], load.reshape(_R, 128))

    sc2 = _make_sc_scatter(N0 // MAXFANOUT, table_in_vmem=True)
    q = sc2(wgidx, wire_indices[0], wire_values, wl.reshape(-1))
    q = q.reshape(NC, _R, 128)
    out = _dense2(q[0], q[1], o, wl)
    return out.reshape(2, L, MAXNODE)
